# baseline (speedup 1.0000x reference)
import functools

import jax
import jax.numpy as jnp
from jax import lax
from jax.experimental import pallas as pl
from jax.experimental.pallas import tpu as pltpu
from jax.experimental.pallas import tpu_sc as plsc

NC = 2
NS = 16
NW = NC * NS
L = 16
CHUNK = 128

_SC_PARAMS = pltpu.CompilerParams(use_tc_tiling_on_sc=False, needs_layout_passes=False)


def _make_deg_kernel(NP, RPT, CPT):
    mesh = plsc.VectorSubcoreMesh(core_axis_name="c", subcore_axis_name="s")

    @functools.partial(
        pl.kernel,
        mesh=mesh,
        out_type=jax.ShapeDtypeStruct((NC, NP, L), jnp.float32),
        scratch_types=[
            pltpu.VMEM_SHARED((NP, L), jnp.float32),
            pltpu.VMEM((CPT, CHUNK), jnp.int32),
            pltpu.VMEM((CHUNK, L), jnp.float32),
        ],
        compiler_params=_SC_PARAMS,
    )
    def deg_kernel(dst_hbm, ones_hbm, zero_hbm, out_hbm, deg_sp, dst_v,
                   ones_v):
        c = lax.axis_index("c")
        s = lax.axis_index("s")
        wid = c * NS + s

        pltpu.sync_copy(zero_hbm.at[pl.ds(s * RPT, RPT)],
                        deg_sp.at[pl.ds(s * RPT, RPT)])
        pltpu.sync_copy(ones_hbm, ones_v)
        pltpu.sync_copy(dst_hbm.at[wid], dst_v)
        plsc.subcore_barrier()

        def edge_body(j, carry):
            pltpu.sync_copy(ones_v, deg_sp.at[dst_v.at[j]], add=True)
            return carry

        lax.fori_loop(0, CPT, edge_body, 0)
        plsc.subcore_barrier()
        pltpu.sync_copy(deg_sp.at[pl.ds(s * RPT, RPT)],
                        out_hbm.at[c, pl.ds(s * RPT, RPT)])

    return deg_kernel


K = 4
NSETS = 2


def _make_agg_kernel(NP, RPT, CPT, W, k=K, spmem_src=False):
    mesh = plsc.VectorSubcoreMesh(core_axis_name="c", subcore_axis_name="s")
    assert CPT % (NSETS * k) == 0
    hsp_scratch = ([pltpu.VMEM_SHARED((NP, W), jnp.float32)]
                   if spmem_src else [])

    @functools.partial(
        pl.kernel,
        mesh=mesh,
        out_type=jax.ShapeDtypeStruct((NC, NP, W), jnp.float32),
        scratch_types=[
            pltpu.VMEM_SHARED((NP, W), jnp.float32),
            *hsp_scratch,
            pltpu.VMEM((CPT, CHUNK), jnp.int32),
            pltpu.VMEM((CPT, CHUNK), jnp.int32),
            pltpu.VMEM((NSETS * k, CHUNK, W), jnp.float32),
            [pltpu.SemaphoreType.DMA] * NSETS,
            [pltpu.SemaphoreType.DMA] * NSETS,
        ],
        compiler_params=_SC_PARAMS,
    )
    def agg_kernel(h_hbm, src_hbm, dst_hbm, zero_hbm, out_hbm,
                   acc_sp, *rest):
        if spmem_src:
            h_sp, src_v, dst_v, rows_v, gsem, ssem = rest
        else:
            src_v, dst_v, rows_v, gsem, ssem = rest
            h_sp = None
        c = lax.axis_index("c")
        s = lax.axis_index("s")
        wid = c * NS + s

        pltpu.sync_copy(zero_hbm.at[pl.ds(s * RPT, RPT)],
                        acc_sp.at[pl.ds(s * RPT, RPT)])
        if spmem_src:
            pltpu.sync_copy(h_hbm.at[pl.ds(s * RPT, RPT)],
                            h_sp.at[pl.ds(s * RPT, RPT)])
        src_ref = h_sp if spmem_src is True else h_hbm
        pltpu.sync_copy(src_hbm.at[wid], src_v)
        pltpu.sync_copy(dst_hbm.at[wid], dst_v)
        plsc.subcore_barrier()

        def gsrc(p):
            return h_sp if (spmem_src == "split" and p == 0) else src_ref

        def start_gather(p, b, j):
            pltpu.async_copy(gsrc(p).at[src_v.at[j]], rows_v.at[p * k + b],
                             gsem[p])

        def drain_gathers(p):
            for b in range(k):
                pltpu.make_async_copy(gsrc(p).at[src_v.at[b]],
                                      rows_v.at[p * k + b], gsem[p]).wait()

        def start_scatter(p, b, j):
            pltpu.async_copy(rows_v.at[p * k + b], acc_sp.at[dst_v.at[j]],
                             ssem[p], add=True)

        def drain_scatters(p):
            for b in range(k):
                pltpu.make_async_copy(rows_v.at[p * k + b],
                                      acc_sp.at[dst_v.at[b]], ssem[p]).wait()

        def edge_body(h, carry):
            g0 = h * NSETS * k
            g1 = g0 + k

            @pl.when(h > 0)
            def _():
                drain_scatters(0)

            for b in range(k):
                start_gather(0, b, g0 + b)

            @pl.when(h > 0)
            def _():
                drain_scatters(1)

            for b in range(k):
                start_gather(1, b, g1 + b)
            drain_gathers(0)
            for b in range(k):
                start_scatter(0, b, g0 + b)
            drain_gathers(1)
            for b in range(k):
                start_scatter(1, b, g1 + b)
            return carry

        lax.fori_loop(0, CPT // (NSETS * k), edge_body, 0)
        drain_scatters(0)
        drain_scatters(1)
        plsc.subcore_barrier()

        pltpu.sync_copy(acc_sp.at[pl.ds(s * RPT, RPT)],
                        out_hbm.at[c, pl.ds(s * RPT, RPT)])

    return agg_kernel


def _dinv_from_partials(degp_ref):
    deg = degp_ref[0, :, 0:1] + degp_ref[1, :, 0:1] + 1.0
    return lax.rsqrt(deg)


def _make_mm1_body(N, NP):
    def mm1_body(x_ref, w_ref, degp_ref, o_ref):
        dinv = _dinv_from_partials(degp_ref)
        h = jnp.dot(x_ref[...], w_ref[...],
                    preferred_element_type=jnp.float32)
        o_ref[pl.ds(0, N), :] = h * dinv[:N]
        o_ref[pl.ds(N, NP - N), :] = jnp.zeros(
            (NP - N, h.shape[1]), jnp.float32)

    return mm1_body


def _mid_body(accp_ref, hs_ref, degp_ref, w2_ref, b1_ref, o_ref):
    dinv = _dinv_from_partials(degp_ref)
    out1 = dinv * (accp_ref[0] + accp_ref[1] + hs_ref[...]) + b1_ref[...]
    z = jnp.maximum(out1, 0.0)
    h2 = jnp.dot(z, w2_ref[...], preferred_element_type=jnp.float32)
    o_ref[...] = h2 * dinv


def _make_fin_body(N, C):
    def fin_body(accp_ref, hs2_ref, degp_ref, b2_ref, o_ref):
        dinv = _dinv_from_partials(degp_ref)
        logits = (dinv * (accp_ref[0] + accp_ref[1] + hs2_ref[...])
                  + b2_ref[...])
        col = lax.broadcasted_iota(jnp.int32, logits.shape, 1)
        valid = col < C
        m = jnp.max(jnp.where(valid, logits, -jnp.inf), axis=1, keepdims=True)
        e = jnp.where(valid, jnp.exp(logits - m), 0.0)
        p = e / jnp.sum(e, axis=1, keepdims=True)
        o_ref[...] = p[:N, :C]

    return fin_body


@jax.jit
def kernel(x, edge_index, W1, b1, W2, b2):
    N, F = x.shape
    H = W1.shape[1]
    C = W2.shape[1]
    E = edge_index.shape[1]

    RPT = -(-(N + 1) // (NS * 8)) * 8
    NP = NS * RPT
    CPT = -(-(-(-E // (NW * CHUNK))) // (NSETS * K)) * (NSETS * K)
    EP = NW * CHUNK * CPT
    W2L = 16

    def pad_edges(e):
        ew = -(-E // NW)
        e = jnp.concatenate(
            [e, jnp.full((NW * ew - E,), N, jnp.int32)]).reshape(NW, ew)
        padw = CPT * CHUNK - ew
        padvals = N + (jnp.arange(padw, dtype=jnp.int32) % (NP - N))
        padblk = jnp.broadcast_to(padvals, (NW, padw))
        return jnp.concatenate([e, padblk], axis=1).reshape(NW, CPT, CHUNK)

    src3 = pad_edges(edge_index[0])
    dst3 = pad_edges(edge_index[1])

    zeros16 = jnp.zeros((NP, W2L), jnp.float32)
    onehot = jnp.zeros((CHUNK, L), jnp.float32).at[:, 0].set(1.0)
    degp = _make_deg_kernel(NP, RPT, CPT)(dst3, onehot, zeros16)

    hs = pl.pallas_call(
        _make_mm1_body(N, NP),
        out_shape=jax.ShapeDtypeStruct((NP, H), jnp.float32),
    )(x, W1, degp)

    accp = _make_agg_kernel(NP, RPT, CPT, H, k=1, spmem_src="split")(
        hs, src3, dst3, jnp.zeros((NP, H), jnp.float32))

    W2p = jnp.pad(W2, ((0, 0), (0, W2L - C)))
    hs2 = pl.pallas_call(
        _mid_body,
        out_shape=jax.ShapeDtypeStruct((NP, W2L), jnp.float32),
    )(accp, hs, degp, W2p, b1[None, :])

    acc2p = _make_agg_kernel(NP, RPT, CPT, W2L)(hs2, src3, dst3, zeros16)

    b2p = jnp.pad(b2, (0, W2L - C))[None, :]
    return pl.pallas_call(
        _make_fin_body(N, C),
        out_shape=jax.ShapeDtypeStruct((N, C), jnp.float32),
    )(acc2p, hs2, degp, b2p)


# speedup vs baseline: 1.1076x; 1.1076x over previous
import functools

import jax
import jax.numpy as jnp
from jax import lax
from jax.experimental import pallas as pl
from jax.experimental.pallas import tpu as pltpu
from jax.experimental.pallas import tpu_sc as plsc

NC = 2
NS = 16
NW = NC * NS
L = 16
CHUNK = 128

_SC_PARAMS = pltpu.CompilerParams(use_tc_tiling_on_sc=False, needs_layout_passes=False)


def _make_deg_kernel(NPD, RPTD, CPT):
    mesh = plsc.VectorSubcoreMesh(core_axis_name="c", subcore_axis_name="s")

    @functools.partial(
        pl.kernel,
        mesh=mesh,
        out_type=jax.ShapeDtypeStruct((NW * RPTD,), jnp.float32),
        scratch_types=[
            pltpu.VMEM_SHARED((NPD,), jnp.float32),
            pltpu.VMEM((CPT, CHUNK), jnp.int32),
            pltpu.VMEM((CHUNK,), jnp.float32),
            pltpu.VMEM((RPTD,), jnp.float32),
        ],
        compiler_params=_SC_PARAMS,
    )
    def deg_kernel(dst_hbm, out_hbm, deg_sp, dst_v, ones_v, stage_v):
        c = lax.axis_index("c")
        s = lax.axis_index("s")
        wid = c * NS + s

        def fill_body(i, carry):
            ones_v[pl.ds(i * L, L)] = jnp.ones((L,), jnp.float32)
            return carry

        lax.fori_loop(0, CHUNK // L, fill_body, 0)

        def zero_body(i, carry):
            stage_v[pl.ds(i * L, L)] = jnp.zeros((L,), jnp.float32)
            return carry

        lax.fori_loop(0, RPTD // L, zero_body, 0)
        pltpu.sync_copy(stage_v, deg_sp.at[pl.ds(s * RPTD, RPTD)])
        pltpu.sync_copy(dst_hbm.at[wid], dst_v)
        plsc.subcore_barrier()

        def edge_body(j, carry):
            pltpu.sync_copy(ones_v, deg_sp.at[dst_v.at[j]], add=True)
            return carry

        lax.fori_loop(0, CPT, edge_body, 0)
        plsc.subcore_barrier()
        pltpu.sync_copy(deg_sp.at[pl.ds(s * RPTD, RPTD)], stage_v)
        pltpu.sync_copy(stage_v, out_hbm.at[pl.ds(wid * RPTD, RPTD)])

    return deg_kernel


K = 4
NSETS = 2


def _make_agg_kernel(NP, RPT, CPT, W, k=K, spmem_src=False):
    mesh = plsc.VectorSubcoreMesh(core_axis_name="c", subcore_axis_name="s")
    assert CPT % (NSETS * k) == 0
    hsp_scratch = ([pltpu.VMEM_SHARED((NP, W), jnp.float32)]
                   if spmem_src else [])

    @functools.partial(
        pl.kernel,
        mesh=mesh,
        out_type=jax.ShapeDtypeStruct((NC, NP, W), jnp.float32),
        scratch_types=[
            pltpu.VMEM_SHARED((NP, W), jnp.float32),
            *hsp_scratch,
            pltpu.VMEM((CPT, CHUNK), jnp.int32),
            pltpu.VMEM((CPT, CHUNK), jnp.int32),
            pltpu.VMEM((NSETS * k, CHUNK, W), jnp.float32),
            [pltpu.SemaphoreType.DMA] * NSETS,
            [pltpu.SemaphoreType.DMA] * NSETS,
        ],
        compiler_params=_SC_PARAMS,
    )
    def agg_kernel(h_hbm, src_hbm, dst_hbm, zero_hbm, out_hbm,
                   acc_sp, *rest):
        if spmem_src:
            h_sp, src_v, dst_v, rows_v, gsem, ssem = rest
        else:
            src_v, dst_v, rows_v, gsem, ssem = rest
            h_sp = None
        c = lax.axis_index("c")
        s = lax.axis_index("s")
        wid = c * NS + s

        pltpu.sync_copy(zero_hbm.at[pl.ds(s * RPT, RPT)],
                        acc_sp.at[pl.ds(s * RPT, RPT)])
        if spmem_src:
            pltpu.sync_copy(h_hbm.at[pl.ds(s * RPT, RPT)],
                            h_sp.at[pl.ds(s * RPT, RPT)])
        src_ref = h_sp if spmem_src is True else h_hbm
        pltpu.sync_copy(src_hbm.at[wid], src_v)
        pltpu.sync_copy(dst_hbm.at[wid], dst_v)
        plsc.subcore_barrier()

        def gsrc(p):
            return h_sp if (spmem_src == "split" and p == 0) else src_ref

        def start_gather(p, b, j):
            pltpu.async_copy(gsrc(p).at[src_v.at[j]], rows_v.at[p * k + b],
                             gsem[p])

        def drain_gathers(p):
            for b in range(k):
                pltpu.make_async_copy(gsrc(p).at[src_v.at[b]],
                                      rows_v.at[p * k + b], gsem[p]).wait()

        def start_scatter(p, b, j):
            pltpu.async_copy(rows_v.at[p * k + b], acc_sp.at[dst_v.at[j]],
                             ssem[p], add=True)

        def drain_scatters(p):
            for b in range(k):
                pltpu.make_async_copy(rows_v.at[p * k + b],
                                      acc_sp.at[dst_v.at[b]], ssem[p]).wait()

        def edge_body(h, carry):
            g0 = h * NSETS * k
            g1 = g0 + k

            @pl.when(h > 0)
            def _():
                drain_scatters(0)

            for b in range(k):
                start_gather(0, b, g0 + b)

            @pl.when(h > 0)
            def _():
                drain_scatters(1)

            for b in range(k):
                start_gather(1, b, g1 + b)
            drain_gathers(0)
            for b in range(k):
                start_scatter(0, b, g0 + b)
            drain_gathers(1)
            for b in range(k):
                start_scatter(1, b, g1 + b)
            return carry

        lax.fori_loop(0, CPT // (NSETS * k), edge_body, 0)
        drain_scatters(0)
        drain_scatters(1)
        plsc.subcore_barrier()

        pltpu.sync_copy(acc_sp.at[pl.ds(s * RPT, RPT)],
                        out_hbm.at[c, pl.ds(s * RPT, RPT)])

    return agg_kernel


def _dinv_from_partials(degp_ref):
    deg = jnp.sum(degp_ref[...], axis=1, keepdims=True) + 1.0
    return lax.rsqrt(deg)


def _make_mm1_body(N, NP):
    def mm1_body(x_ref, w_ref, degp_ref, o_ref):
        dinv = _dinv_from_partials(degp_ref)
        h = jnp.dot(x_ref[...], w_ref[...],
                    preferred_element_type=jnp.float32)
        o_ref[pl.ds(0, N), :] = h * dinv[:N]
        o_ref[pl.ds(N, NP - N), :] = jnp.zeros(
            (NP - N, h.shape[1]), jnp.float32)

    return mm1_body


def _mid_body(accp_ref, hs_ref, degp_ref, w2_ref, b1_ref, o_ref):
    dinv = _dinv_from_partials(degp_ref)
    out1 = dinv * (accp_ref[0] + accp_ref[1] + hs_ref[...]) + b1_ref[...]
    z = jnp.maximum(out1, 0.0)
    h2 = jnp.dot(z, w2_ref[...], preferred_element_type=jnp.float32)
    o_ref[...] = h2 * dinv


def _make_fin_body(N, C):
    def fin_body(accp_ref, hs2_ref, degp_ref, b2_ref, o_ref):
        dinv = _dinv_from_partials(degp_ref)
        logits = (dinv * (accp_ref[0] + accp_ref[1] + hs2_ref[...])
                  + b2_ref[...])
        col = lax.broadcasted_iota(jnp.int32, logits.shape, 1)
        valid = col < C
        m = jnp.max(jnp.where(valid, logits, -jnp.inf), axis=1, keepdims=True)
        e = jnp.where(valid, jnp.exp(logits - m), 0.0)
        p = e / jnp.sum(e, axis=1, keepdims=True)
        o_ref[...] = p[:N, :C]

    return fin_body


@jax.jit
def kernel(x, edge_index, W1, b1, W2, b2):
    N, F = x.shape
    H = W1.shape[1]
    C = W2.shape[1]
    E = edge_index.shape[1]

    RPT = -(-(N + 1) // (NS * 8)) * 8
    NP = NS * RPT
    CPT = -(-(-(-E // (NW * CHUNK))) // (NSETS * K)) * (NSETS * K)
    EP = NW * CHUNK * CPT
    W2L = 16

    def pad_edges(e):
        ew = -(-E // NW)
        e = jnp.concatenate(
            [e, jnp.full((NW * ew - E,), N, jnp.int32)]).reshape(NW, ew)
        padw = CPT * CHUNK - ew
        padvals = N + (jnp.arange(padw, dtype=jnp.int32) % (NP - N))
        padblk = jnp.broadcast_to(padvals, (NW, padw))
        return jnp.concatenate([e, padblk], axis=1).reshape(NW, CPT, CHUNK)

    src3 = pad_edges(edge_index[0])
    dst3 = pad_edges(edge_index[1])

    RPTD = -(-(N + 1) // (NS * 128)) * 128
    NPD = NS * RPTD
    zeros16 = jnp.zeros((NP, W2L), jnp.float32)
    degp = _make_deg_kernel(NPD, RPTD, CPT)(dst3)
    degp = degp.reshape(NC, NPD)[:, :NP].T

    hs = pl.pallas_call(
        _make_mm1_body(N, NP),
        out_shape=jax.ShapeDtypeStruct((NP, H), jnp.float32),
    )(x, W1, degp)

    accp = _make_agg_kernel(NP, RPT, CPT, H)(
        hs, src3, dst3, jnp.zeros((NP, H), jnp.float32))

    W2p = jnp.pad(W2, ((0, 0), (0, W2L - C)))
    hs2 = pl.pallas_call(
        _mid_body,
        out_shape=jax.ShapeDtypeStruct((NP, W2L), jnp.float32),
    )(accp, hs, degp, W2p, b1[None, :])

    acc2p = _make_agg_kernel(NP, RPT, CPT, W2L)(hs2, src3, dst3, zeros16)

    b2p = jnp.pad(b2, (0, W2L - C))[None, :]
    return pl.pallas_call(
        _make_fin_body(N, C),
        out_shape=jax.ShapeDtypeStruct((N, C), jnp.float32),
    )(acc2p, hs2, degp, b2p)


# speedup vs baseline: 1.1141x; 1.0059x over previous
import functools

import jax
import jax.numpy as jnp
from jax import lax
from jax.experimental import pallas as pl
from jax.experimental.pallas import tpu as pltpu
from jax.experimental.pallas import tpu_sc as plsc

NC = 2
NS = 16
NW = NC * NS
L = 16
CHUNK = 128

_SC_PARAMS = pltpu.CompilerParams(use_tc_tiling_on_sc=False, needs_layout_passes=False)


def _make_deg_kernel(NPD, RPTD, CPT):
    mesh = plsc.VectorSubcoreMesh(core_axis_name="c", subcore_axis_name="s")

    @functools.partial(
        pl.kernel,
        mesh=mesh,
        out_type=jax.ShapeDtypeStruct((NW * RPTD,), jnp.float32),
        scratch_types=[
            pltpu.VMEM_SHARED((NPD,), jnp.float32),
            pltpu.VMEM((CPT, CHUNK), jnp.int32),
            pltpu.VMEM((CHUNK,), jnp.float32),
            pltpu.VMEM((RPTD,), jnp.float32),
        ],
        compiler_params=_SC_PARAMS,
    )
    def deg_kernel(dst_hbm, out_hbm, deg_sp, dst_v, ones_v, stage_v):
        c = lax.axis_index("c")
        s = lax.axis_index("s")
        wid = c * NS + s

        def fill_body(i, carry):
            ones_v[pl.ds(i * L, L)] = jnp.ones((L,), jnp.float32)
            return carry

        lax.fori_loop(0, CHUNK // L, fill_body, 0)

        def zero_body(i, carry):
            stage_v[pl.ds(i * L, L)] = jnp.zeros((L,), jnp.float32)
            return carry

        lax.fori_loop(0, RPTD // L, zero_body, 0)
        pltpu.sync_copy(stage_v, deg_sp.at[pl.ds(s * RPTD, RPTD)])
        pltpu.sync_copy(dst_hbm.at[wid], dst_v)
        plsc.subcore_barrier()

        def edge_body(j, carry):
            pltpu.sync_copy(ones_v, deg_sp.at[dst_v.at[j]], add=True)
            return carry

        lax.fori_loop(0, CPT, edge_body, 0)
        plsc.subcore_barrier()
        pltpu.sync_copy(deg_sp.at[pl.ds(s * RPTD, RPTD)], stage_v)
        pltpu.sync_copy(stage_v, out_hbm.at[pl.ds(wid * RPTD, RPTD)])

    return deg_kernel


K = 4
NSETS = 2


def _make_agg_kernel(NP, RPT, CPT, W, k=K, spmem_src=False):
    mesh = plsc.VectorSubcoreMesh(core_axis_name="c", subcore_axis_name="s")
    assert CPT % (NSETS * k) == 0
    hsp_scratch = ([pltpu.VMEM_SHARED((NP, W), jnp.float32)]
                   if spmem_src else [])

    @functools.partial(
        pl.kernel,
        mesh=mesh,
        out_type=jax.ShapeDtypeStruct((NC, NP, W), jnp.float32),
        scratch_types=[
            pltpu.VMEM_SHARED((NP, W), jnp.float32),
            *hsp_scratch,
            pltpu.VMEM((CPT, CHUNK), jnp.int32),
            pltpu.VMEM((CPT, CHUNK), jnp.int32),
            pltpu.VMEM((NSETS * k, CHUNK, W), jnp.float32),
            [pltpu.SemaphoreType.DMA] * NSETS,
            [pltpu.SemaphoreType.DMA] * NSETS,
        ],
        compiler_params=_SC_PARAMS,
    )
    def agg_kernel(h_hbm, src_hbm, dst_hbm, zero_hbm, out_hbm,
                   acc_sp, *rest):
        if spmem_src:
            h_sp, src_v, dst_v, rows_v, gsem, ssem = rest
        else:
            src_v, dst_v, rows_v, gsem, ssem = rest
            h_sp = None
        c = lax.axis_index("c")
        s = lax.axis_index("s")
        wid = c * NS + s

        pltpu.sync_copy(zero_hbm.at[pl.ds(s * RPT, RPT)],
                        acc_sp.at[pl.ds(s * RPT, RPT)])
        if spmem_src:
            pltpu.sync_copy(h_hbm.at[pl.ds(s * RPT, RPT)],
                            h_sp.at[pl.ds(s * RPT, RPT)])
        src_ref = h_sp if spmem_src is True else h_hbm
        pltpu.sync_copy(src_hbm.at[wid], src_v)
        pltpu.sync_copy(dst_hbm.at[wid], dst_v)
        plsc.subcore_barrier()

        def gsrc(p):
            return h_sp if (spmem_src == "split" and p == 0) else src_ref

        def start_gather(p, b, j):
            pltpu.async_copy(gsrc(p).at[src_v.at[j]], rows_v.at[p * k + b],
                             gsem[p])

        def drain_gathers(p):
            for b in range(k):
                pltpu.make_async_copy(gsrc(p).at[src_v.at[b]],
                                      rows_v.at[p * k + b], gsem[p]).wait()

        def start_scatter(p, b, j):
            pltpu.async_copy(rows_v.at[p * k + b], acc_sp.at[dst_v.at[j]],
                             ssem[p], add=True)

        def drain_scatters(p):
            for b in range(k):
                pltpu.make_async_copy(rows_v.at[p * k + b],
                                      acc_sp.at[dst_v.at[b]], ssem[p]).wait()

        def edge_body(h, carry):
            g0 = h * NSETS * k
            g1 = g0 + k

            @pl.when(h > 0)
            def _():
                drain_scatters(0)

            for b in range(k):
                start_gather(0, b, g0 + b)

            @pl.when(h > 0)
            def _():
                drain_scatters(1)

            for b in range(k):
                start_gather(1, b, g1 + b)
            drain_gathers(0)
            for b in range(k):
                start_scatter(0, b, g0 + b)
            drain_gathers(1)
            for b in range(k):
                start_scatter(1, b, g1 + b)
            return carry

        lax.fori_loop(0, CPT // (NSETS * k), edge_body, 0)
        drain_scatters(0)
        drain_scatters(1)
        plsc.subcore_barrier()

        pltpu.sync_copy(acc_sp.at[pl.ds(s * RPT, RPT)],
                        out_hbm.at[c, pl.ds(s * RPT, RPT)])

    return agg_kernel


def _dinv_from_partials(degp_ref):
    deg = jnp.sum(degp_ref[...], axis=1, keepdims=True) + 1.0
    return lax.rsqrt(deg)


def _make_mm1_body(N, NP):
    def mm1_body(x_ref, w_ref, degp_ref, o_ref):
        dinv = _dinv_from_partials(degp_ref)
        h = jnp.dot(x_ref[...], w_ref[...],
                    preferred_element_type=jnp.float32)
        o_ref[pl.ds(0, N), :] = h * dinv[:N]
        o_ref[pl.ds(N, NP - N), :] = jnp.zeros(
            (NP - N, h.shape[1]), jnp.float32)

    return mm1_body


def _mid_body(accp_ref, hs_ref, degp_ref, w2_ref, b1_ref, o_ref):
    dinv = _dinv_from_partials(degp_ref)
    out1 = dinv * (accp_ref[0] + accp_ref[1] + hs_ref[...]) + b1_ref[...]
    z = jnp.maximum(out1, 0.0)
    h2 = jnp.dot(z, w2_ref[...], preferred_element_type=jnp.float32)
    o_ref[...] = h2 * dinv


def _make_fin_body(N, C):
    def fin_body(accp_ref, hs2_ref, degp_ref, b2_ref, o_ref):
        dinv = _dinv_from_partials(degp_ref)
        logits = (dinv * (accp_ref[0] + accp_ref[1] + hs2_ref[...])
                  + b2_ref[...])
        col = lax.broadcasted_iota(jnp.int32, logits.shape, 1)
        valid = col < C
        m = jnp.max(jnp.where(valid, logits, -jnp.inf), axis=1, keepdims=True)
        e = jnp.where(valid, jnp.exp(logits - m), 0.0)
        p = e / jnp.sum(e, axis=1, keepdims=True)
        o_ref[...] = p[:N, :C]

    return fin_body


@jax.jit
def kernel(x, edge_index, W1, b1, W2, b2):
    N, F = x.shape
    H = W1.shape[1]
    C = W2.shape[1]
    E = edge_index.shape[1]

    RPT = -(-(N + 1) // (NS * 8)) * 8
    NP = NS * RPT
    CPT = -(-(-(-E // (NW * CHUNK))) // (NSETS * K)) * (NSETS * K)
    EP = NW * CHUNK * CPT
    W2L = 16

    def pad_edges(e):
        ew = -(-E // NW)
        e = jnp.concatenate(
            [e, jnp.full((NW * ew - E,), N, jnp.int32)]).reshape(NW, ew)
        padw = CPT * CHUNK - ew
        padvals = N + (jnp.arange(padw, dtype=jnp.int32) % (NP - N))
        padblk = jnp.broadcast_to(padvals, (NW, padw))
        return jnp.concatenate([e, padblk], axis=1).reshape(NW, CPT, CHUNK)

    src3 = pad_edges(edge_index[0])
    dst3 = pad_edges(edge_index[1])

    RPTD = -(-(N + 1) // (NS * 128)) * 128
    NPD = NS * RPTD
    zeros16 = jnp.zeros((NP, W2L), jnp.float32)
    degp = _make_deg_kernel(NPD, RPTD, CPT)(dst3)
    degp = degp.reshape(NC, NPD)[:, :NP].T

    hs = pl.pallas_call(
        _make_mm1_body(N, NP),
        out_shape=jax.ShapeDtypeStruct((NP, H), jnp.float32),
    )(x, W1, degp)

    accp = _make_agg_kernel(NP, RPT, CPT, H)(
        hs, src3, dst3, jnp.zeros((NP, H), jnp.float32))

    W2p = jnp.pad(W2, ((0, 0), (0, W2L - C)))
    hs2 = pl.pallas_call(
        _mid_body,
        out_shape=jax.ShapeDtypeStruct((NP, W2L), jnp.float32),
    )(accp, hs, degp, W2p, b1[None, :])

    acc2p = _make_agg_kernel(NP, RPT, CPT, W2L, k=8)(
        hs2, src3, dst3, zeros16)

    b2p = jnp.pad(b2, (0, W2L - C))[None, :]
    return pl.pallas_call(
        _make_fin_body(N, C),
        out_shape=jax.ShapeDtypeStruct((N, C), jnp.float32),
    )(acc2p, hs2, degp, b2p)


# speedup vs baseline: 1.1367x; 1.0202x over previous
import functools

import jax
import jax.numpy as jnp
from jax import lax
from jax.experimental import pallas as pl
from jax.experimental.pallas import tpu as pltpu
from jax.experimental.pallas import tpu_sc as plsc

NC = 2
NS = 16
NW = NC * NS
L = 16
CHUNK = 128

_SC_PARAMS = pltpu.CompilerParams(use_tc_tiling_on_sc=False, needs_layout_passes=False)


def _make_deg_kernel(NPD, RPTD, CPT):
    mesh = plsc.VectorSubcoreMesh(core_axis_name="c", subcore_axis_name="s")

    @functools.partial(
        pl.kernel,
        mesh=mesh,
        out_type=jax.ShapeDtypeStruct((NW * RPTD,), jnp.float32),
        scratch_types=[
            pltpu.VMEM_SHARED((NPD,), jnp.float32),
            pltpu.VMEM((CPT, CHUNK), jnp.int32),
            pltpu.VMEM((CHUNK,), jnp.float32),
            pltpu.VMEM((RPTD,), jnp.float32),
            pltpu.SemaphoreType.DMA,
        ],
        compiler_params=_SC_PARAMS,
    )
    def deg_kernel(dst_hbm, out_hbm, deg_sp, dst_v, ones_v, stage_v, sem):
        c = lax.axis_index("c")
        s = lax.axis_index("s")
        wid = c * NS + s

        def fill_body(i, carry):
            ones_v[pl.ds(i * L, L)] = jnp.ones((L,), jnp.float32)
            return carry

        lax.fori_loop(0, CHUNK // L, fill_body, 0)

        def zero_body(i, carry):
            stage_v[pl.ds(i * L, L)] = jnp.zeros((L,), jnp.float32)
            return carry

        lax.fori_loop(0, RPTD // L, zero_body, 0)
        pltpu.sync_copy(stage_v, deg_sp.at[pl.ds(s * RPTD, RPTD)])
        pltpu.sync_copy(dst_hbm.at[wid], dst_v)
        plsc.subcore_barrier()

        DG = 8

        def edge_body(g, carry):
            for b in range(DG):
                pltpu.async_copy(ones_v, deg_sp.at[dst_v.at[g * DG + b]],
                                 sem, add=True)
            for b in range(DG):
                pltpu.make_async_copy(ones_v, deg_sp.at[dst_v.at[b]],
                                      sem).wait()
            return carry

        lax.fori_loop(0, CPT // DG, edge_body, 0)
        plsc.subcore_barrier()
        pltpu.sync_copy(deg_sp.at[pl.ds(s * RPTD, RPTD)], stage_v)
        pltpu.sync_copy(stage_v, out_hbm.at[pl.ds(wid * RPTD, RPTD)])

    return deg_kernel


K = 4
NSETS = 2


def _make_agg_kernel(NP, RPT, CPT, W, k=K, spmem_src=False):
    mesh = plsc.VectorSubcoreMesh(core_axis_name="c", subcore_axis_name="s")
    assert CPT % (NSETS * k) == 0
    hsp_scratch = ([pltpu.VMEM_SHARED((NP, W), jnp.float32)]
                   if spmem_src else [])

    @functools.partial(
        pl.kernel,
        mesh=mesh,
        out_type=jax.ShapeDtypeStruct((NC, NP, W), jnp.float32),
        scratch_types=[
            pltpu.VMEM_SHARED((NP, W), jnp.float32),
            *hsp_scratch,
            pltpu.VMEM((CPT, CHUNK), jnp.int32),
            pltpu.VMEM((CPT, CHUNK), jnp.int32),
            pltpu.VMEM((NSETS * k, CHUNK, W), jnp.float32),
            [pltpu.SemaphoreType.DMA] * NSETS,
            [pltpu.SemaphoreType.DMA] * NSETS,
        ],
        compiler_params=_SC_PARAMS,
    )
    def agg_kernel(h_hbm, src_hbm, dst_hbm, zero_hbm, out_hbm,
                   acc_sp, *rest):
        if spmem_src:
            h_sp, src_v, dst_v, rows_v, gsem, ssem = rest
        else:
            src_v, dst_v, rows_v, gsem, ssem = rest
            h_sp = None
        c = lax.axis_index("c")
        s = lax.axis_index("s")
        wid = c * NS + s

        pltpu.sync_copy(zero_hbm.at[pl.ds(s * RPT, RPT)],
                        acc_sp.at[pl.ds(s * RPT, RPT)])
        if spmem_src:
            pltpu.sync_copy(h_hbm.at[pl.ds(s * RPT, RPT)],
                            h_sp.at[pl.ds(s * RPT, RPT)])
        src_ref = h_sp if spmem_src is True else h_hbm
        pltpu.sync_copy(src_hbm.at[wid], src_v)
        pltpu.sync_copy(dst_hbm.at[wid], dst_v)
        plsc.subcore_barrier()

        def gsrc(p):
            return h_sp if (spmem_src == "split" and p == 0) else src_ref

        def start_gather(p, b, j):
            pltpu.async_copy(gsrc(p).at[src_v.at[j]], rows_v.at[p * k + b],
                             gsem[p])

        def drain_gathers(p):
            for b in range(k):
                pltpu.make_async_copy(gsrc(p).at[src_v.at[b]],
                                      rows_v.at[p * k + b], gsem[p]).wait()

        def start_scatter(p, b, j):
            pltpu.async_copy(rows_v.at[p * k + b], acc_sp.at[dst_v.at[j]],
                             ssem[p], add=True)

        def drain_scatters(p):
            for b in range(k):
                pltpu.make_async_copy(rows_v.at[p * k + b],
                                      acc_sp.at[dst_v.at[b]], ssem[p]).wait()

        def edge_body(h, carry):
            g0 = h * NSETS * k
            g1 = g0 + k

            @pl.when(h > 0)
            def _():
                drain_scatters(0)

            for b in range(k):
                start_gather(0, b, g0 + b)

            @pl.when(h > 0)
            def _():
                drain_scatters(1)

            for b in range(k):
                start_gather(1, b, g1 + b)
            drain_gathers(0)
            for b in range(k):
                start_scatter(0, b, g0 + b)
            drain_gathers(1)
            for b in range(k):
                start_scatter(1, b, g1 + b)
            return carry

        lax.fori_loop(0, CPT // (NSETS * k), edge_body, 0)
        drain_scatters(0)
        drain_scatters(1)
        plsc.subcore_barrier()

        pltpu.sync_copy(acc_sp.at[pl.ds(s * RPT, RPT)],
                        out_hbm.at[c, pl.ds(s * RPT, RPT)])

    return agg_kernel


def _dinv_from_partials(degp_ref):
    deg = jnp.sum(degp_ref[...], axis=1, keepdims=True) + 1.0
    return lax.rsqrt(deg)


def _make_mm1_body(N, NP):
    def mm1_body(x_ref, w_ref, degp_ref, o_ref):
        dinv = _dinv_from_partials(degp_ref)
        h = jnp.dot(x_ref[...], w_ref[...],
                    preferred_element_type=jnp.float32)
        o_ref[pl.ds(0, N), :] = h * dinv[:N]
        o_ref[pl.ds(N, NP - N), :] = jnp.zeros(
            (NP - N, h.shape[1]), jnp.float32)

    return mm1_body


def _mid_body(accp_ref, hs_ref, degp_ref, w2_ref, b1_ref, o_ref):
    dinv = _dinv_from_partials(degp_ref)
    out1 = dinv * (accp_ref[0] + accp_ref[1] + hs_ref[...]) + b1_ref[...]
    z = jnp.maximum(out1, 0.0)
    h2 = jnp.dot(z, w2_ref[...], preferred_element_type=jnp.float32)
    o_ref[...] = h2 * dinv


def _make_fin_body(N, C):
    def fin_body(accp_ref, hs2_ref, degp_ref, b2_ref, o_ref):
        dinv = _dinv_from_partials(degp_ref)
        logits = (dinv * (accp_ref[0] + accp_ref[1] + hs2_ref[...])
                  + b2_ref[...])
        col = lax.broadcasted_iota(jnp.int32, logits.shape, 1)
        valid = col < C
        m = jnp.max(jnp.where(valid, logits, -jnp.inf), axis=1, keepdims=True)
        e = jnp.where(valid, jnp.exp(logits - m), 0.0)
        p = e / jnp.sum(e, axis=1, keepdims=True)
        o_ref[...] = p[:N, :C]

    return fin_body


@jax.jit
def kernel(x, edge_index, W1, b1, W2, b2):
    N, F = x.shape
    H = W1.shape[1]
    C = W2.shape[1]
    E = edge_index.shape[1]

    RPT = -(-(N + 1) // (NS * 8)) * 8
    NP = NS * RPT
    CPT = -(-(-(-E // (NW * CHUNK))) // (NSETS * K)) * (NSETS * K)
    EP = NW * CHUNK * CPT
    W2L = 16

    def pad_edges(e):
        ew = -(-E // NW)
        e = jnp.concatenate(
            [e, jnp.full((NW * ew - E,), N, jnp.int32)]).reshape(NW, ew)
        padw = CPT * CHUNK - ew
        padvals = N + (jnp.arange(padw, dtype=jnp.int32) % (NP - N))
        padblk = jnp.broadcast_to(padvals, (NW, padw))
        return jnp.concatenate([e, padblk], axis=1).reshape(NW, CPT, CHUNK)

    src3 = pad_edges(edge_index[0])
    dst3 = pad_edges(edge_index[1])

    RPTD = -(-(N + 1) // (NS * 128)) * 128
    NPD = NS * RPTD
    zeros16 = jnp.zeros((NP, W2L), jnp.float32)
    degp = _make_deg_kernel(NPD, RPTD, CPT)(dst3)
    degp = degp.reshape(NC, NPD)[:, :NP].T

    hs = pl.pallas_call(
        _make_mm1_body(N, NP),
        out_shape=jax.ShapeDtypeStruct((NP, H), jnp.float32),
    )(x, W1, degp)

    accp = _make_agg_kernel(NP, RPT, CPT, H)(
        hs, src3, dst3, jnp.zeros((NP, H), jnp.float32))

    W2p = jnp.pad(W2, ((0, 0), (0, W2L - C)))
    hs2 = pl.pallas_call(
        _mid_body,
        out_shape=jax.ShapeDtypeStruct((NP, W2L), jnp.float32),
    )(accp, hs, degp, W2p, b1[None, :])

    acc2p = _make_agg_kernel(NP, RPT, CPT, W2L, k=8)(
        hs2, src3, dst3, zeros16)

    b2p = jnp.pad(b2, (0, W2L - C))[None, :]
    return pl.pallas_call(
        _make_fin_body(N, C),
        out_shape=jax.ShapeDtypeStruct((N, C), jnp.float32),
    )(acc2p, hs2, degp, b2p)


# speedup vs baseline: 1.1567x; 1.0177x over previous
import functools

import jax
import jax.numpy as jnp
from jax import lax
from jax.experimental import pallas as pl
from jax.experimental.pallas import tpu as pltpu
from jax.experimental.pallas import tpu_sc as plsc

NC = 2
NS = 16
NW = NC * NS
L = 16
CHUNK = 128

_SC_PARAMS = pltpu.CompilerParams(use_tc_tiling_on_sc=False, needs_layout_passes=False)


def _make_deg_kernel(NPD, RPTD, CPT):
    mesh = plsc.VectorSubcoreMesh(core_axis_name="c", subcore_axis_name="s")

    @functools.partial(
        pl.kernel,
        mesh=mesh,
        out_type=jax.ShapeDtypeStruct((NW * RPTD,), jnp.float32),
        scratch_types=[
            pltpu.VMEM_SHARED((NPD,), jnp.float32),
            pltpu.VMEM((CPT, CHUNK), jnp.int32),
            pltpu.VMEM((CHUNK,), jnp.float32),
            pltpu.VMEM((RPTD,), jnp.float32),
            pltpu.SemaphoreType.DMA,
        ],
        compiler_params=_SC_PARAMS,
    )
    def deg_kernel(dst_hbm, out_hbm, deg_sp, dst_v, ones_v, stage_v, sem):
        c = lax.axis_index("c")
        s = lax.axis_index("s")
        wid = c * NS + s

        def fill_body(i, carry):
            ones_v[pl.ds(i * L, L)] = jnp.ones((L,), jnp.float32)
            return carry

        lax.fori_loop(0, CHUNK // L, fill_body, 0)

        def zero_body(i, carry):
            stage_v[pl.ds(i * L, L)] = jnp.zeros((L,), jnp.float32)
            return carry

        lax.fori_loop(0, RPTD // L, zero_body, 0)
        pltpu.sync_copy(stage_v, deg_sp.at[pl.ds(s * RPTD, RPTD)])
        pltpu.sync_copy(dst_hbm.at[wid], dst_v)
        plsc.subcore_barrier()

        DG = 8

        def edge_body(g, carry):
            for b in range(DG):
                pltpu.async_copy(ones_v, deg_sp.at[dst_v.at[g * DG + b]],
                                 sem, add=True)
            for b in range(DG):
                pltpu.make_async_copy(ones_v, deg_sp.at[dst_v.at[b]],
                                      sem).wait()
            return carry

        lax.fori_loop(0, CPT // DG, edge_body, 0)
        plsc.subcore_barrier()
        pltpu.sync_copy(deg_sp.at[pl.ds(s * RPTD, RPTD)], stage_v)
        pltpu.sync_copy(stage_v, out_hbm.at[pl.ds(wid * RPTD, RPTD)])

    return deg_kernel


K = 4
NSETS = 2


def _make_agg_kernel(NP, RPT, CPT, W, k=K, spmem_src=False):
    mesh = plsc.VectorSubcoreMesh(core_axis_name="c", subcore_axis_name="s")
    assert CPT % (NSETS * k) == 0
    hsp_scratch = ([pltpu.VMEM_SHARED((NP, W), jnp.float32)]
                   if spmem_src else [])

    @functools.partial(
        pl.kernel,
        mesh=mesh,
        out_type=jax.ShapeDtypeStruct((NC, NP, W), jnp.float32),
        scratch_types=[
            pltpu.VMEM_SHARED((NP, W), jnp.float32),
            *hsp_scratch,
            pltpu.VMEM((CPT, CHUNK), jnp.int32),
            pltpu.VMEM((CPT, CHUNK), jnp.int32),
            pltpu.VMEM((NSETS * k, CHUNK, W), jnp.float32),
            [pltpu.SemaphoreType.DMA] * NSETS,
            [pltpu.SemaphoreType.DMA] * NSETS,
        ],
        compiler_params=_SC_PARAMS,
    )
    def agg_kernel(h_hbm, src_hbm, dst_hbm, zero_hbm, out_hbm,
                   acc_sp, *rest):
        if spmem_src:
            h_sp, src_v, dst_v, rows_v, gsem, ssem = rest
        else:
            src_v, dst_v, rows_v, gsem, ssem = rest
            h_sp = None
        c = lax.axis_index("c")
        s = lax.axis_index("s")
        wid = c * NS + s

        prologue = [
            (zero_hbm.at[pl.ds(s * RPT, RPT)], acc_sp.at[pl.ds(s * RPT, RPT)]),
            (src_hbm.at[wid], src_v),
            (dst_hbm.at[wid], dst_v),
        ]
        if spmem_src:
            prologue.append((h_hbm.at[pl.ds(s * RPT, RPT)],
                             h_sp.at[pl.ds(s * RPT, RPT)]))
        src_ref = h_sp if spmem_src is True else h_hbm
        for a, b_ in prologue:
            pltpu.async_copy(a, b_, gsem[0])
        for a, b_ in prologue:
            pltpu.make_async_copy(a, b_, gsem[0]).wait()
        plsc.subcore_barrier()

        def gsrc(p):
            return h_sp if (spmem_src == "split" and p == 0) else src_ref

        def start_gather(p, b, j):
            pltpu.async_copy(gsrc(p).at[src_v.at[j]], rows_v.at[p * k + b],
                             gsem[p])

        def drain_gathers(p):
            for b in range(k):
                pltpu.make_async_copy(gsrc(p).at[src_v.at[b]],
                                      rows_v.at[p * k + b], gsem[p]).wait()

        def start_scatter(p, b, j):
            pltpu.async_copy(rows_v.at[p * k + b], acc_sp.at[dst_v.at[j]],
                             ssem[p], add=True)

        def drain_scatters(p):
            for b in range(k):
                pltpu.make_async_copy(rows_v.at[p * k + b],
                                      acc_sp.at[dst_v.at[b]], ssem[p]).wait()

        def edge_body(h, carry):
            g0 = h * NSETS * k
            g1 = g0 + k

            @pl.when(h > 0)
            def _():
                drain_scatters(0)

            for b in range(k):
                start_gather(0, b, g0 + b)

            @pl.when(h > 0)
            def _():
                drain_scatters(1)

            for b in range(k):
                start_gather(1, b, g1 + b)
            drain_gathers(0)
            for b in range(k):
                start_scatter(0, b, g0 + b)
            drain_gathers(1)
            for b in range(k):
                start_scatter(1, b, g1 + b)
            return carry

        lax.fori_loop(0, CPT // (NSETS * k), edge_body, 0)
        drain_scatters(0)
        drain_scatters(1)
        plsc.subcore_barrier()

        pltpu.sync_copy(acc_sp.at[pl.ds(s * RPT, RPT)],
                        out_hbm.at[c, pl.ds(s * RPT, RPT)])

    return agg_kernel


def _dinv_from_partials(degp_ref):
    deg = jnp.sum(degp_ref[...], axis=1, keepdims=True) + 1.0
    return lax.rsqrt(deg)


def _make_mm1_body(N, NP):
    def mm1_body(x_ref, w_ref, degp_ref, o_ref):
        dinv = _dinv_from_partials(degp_ref)
        h = jnp.dot(x_ref[...], w_ref[...],
                    preferred_element_type=jnp.float32)
        o_ref[pl.ds(0, N), :] = h * dinv[:N]
        o_ref[pl.ds(N, NP - N), :] = jnp.zeros(
            (NP - N, h.shape[1]), jnp.float32)

    return mm1_body


def _mid_body(accp_ref, hs_ref, degp_ref, w2_ref, b1_ref, o_ref):
    dinv = _dinv_from_partials(degp_ref)
    out1 = dinv * (accp_ref[0] + accp_ref[1] + hs_ref[...]) + b1_ref[...]
    z = jnp.maximum(out1, 0.0)
    h2 = jnp.dot(z, w2_ref[...], preferred_element_type=jnp.float32)
    o_ref[...] = h2 * dinv


def _make_fin_body(N, C):
    def fin_body(accp_ref, hs2_ref, degp_ref, b2_ref, o_ref):
        dinv = _dinv_from_partials(degp_ref)
        logits = (dinv * (accp_ref[0] + accp_ref[1] + hs2_ref[...])
                  + b2_ref[...])
        col = lax.broadcasted_iota(jnp.int32, logits.shape, 1)
        valid = col < C
        m = jnp.max(jnp.where(valid, logits, -jnp.inf), axis=1, keepdims=True)
        e = jnp.where(valid, jnp.exp(logits - m), 0.0)
        p = e / jnp.sum(e, axis=1, keepdims=True)
        o_ref[...] = p[:N, :C]

    return fin_body


@jax.jit
def kernel(x, edge_index, W1, b1, W2, b2):
    N, F = x.shape
    H = W1.shape[1]
    C = W2.shape[1]
    E = edge_index.shape[1]

    RPT = -(-(N + 1) // (NS * 8)) * 8
    NP = NS * RPT
    CPT = -(-(-(-E // (NW * CHUNK))) // (NSETS * K)) * (NSETS * K)
    EP = NW * CHUNK * CPT
    W2L = 16

    def pad_edges(e):
        ew = -(-E // NW)
        e = jnp.concatenate(
            [e, jnp.full((NW * ew - E,), N, jnp.int32)]).reshape(NW, ew)
        padw = CPT * CHUNK - ew
        padvals = N + (jnp.arange(padw, dtype=jnp.int32) % (NP - N))
        padblk = jnp.broadcast_to(padvals, (NW, padw))
        return jnp.concatenate([e, padblk], axis=1).reshape(NW, CPT, CHUNK)

    src3 = pad_edges(edge_index[0])
    dst3 = pad_edges(edge_index[1])

    RPTD = -(-(N + 1) // (NS * 128)) * 128
    NPD = NS * RPTD
    zeros16 = jnp.zeros((NP, W2L), jnp.float32)
    degp = _make_deg_kernel(NPD, RPTD, CPT)(dst3)
    degp = degp.reshape(NC, NPD)[:, :NP].T

    hs = pl.pallas_call(
        _make_mm1_body(N, NP),
        out_shape=jax.ShapeDtypeStruct((NP, H), jnp.float32),
    )(x, W1, degp)

    accp = _make_agg_kernel(NP, RPT, CPT, H)(
        hs, src3, dst3, jnp.zeros((NP, H), jnp.float32))

    W2p = jnp.pad(W2, ((0, 0), (0, W2L - C)))
    hs2 = pl.pallas_call(
        _mid_body,
        out_shape=jax.ShapeDtypeStruct((NP, W2L), jnp.float32),
    )(accp, hs, degp, W2p, b1[None, :])

    acc2p = _make_agg_kernel(NP, RPT, CPT, W2L, k=8)(
        hs2, src3, dst3, zeros16)

    b2p = jnp.pad(b2, (0, W2L - C))[None, :]
    return pl.pallas_call(
        _make_fin_body(N, C),
        out_shape=jax.ShapeDtypeStruct((N, C), jnp.float32),
    )(acc2p, hs2, degp, b2p)
